# baseline (speedup 1.0000x reference)
import jax
import jax.numpy as jnp
from jax.experimental import pallas as pl
from jax.experimental.pallas import tpu as pltpu
from jax.experimental.pallas import tpu_sc as plsc

B, T, D = 1, 2048, 768
FF = 1536
E = 8
N = B * T
K = 2
NK = N * K

BLK = 512
NBLK = NK // BLK + E
PAD_N = NBLK * BLK
GW = 128
BT = 256


def _silu(v):
    return v * jax.nn.sigmoid(v)


def _dot(a, b):
    return jax.lax.dot_general(a, b, (((1,), (0,)), ((), ())),
                               preferred_element_type=jnp.float32)


def _sc_gather(table, indices):
    SPLIT = 2
    R, W0 = table.shape
    table = table.reshape(R * SPLIT, W0 // SPLIT)
    indices = (indices[:, None] * SPLIT +
               jnp.arange(SPLIT, dtype=jnp.int32)[None, :]).reshape(-1)
    M = indices.shape[0]
    W = table.shape[1]
    idx2 = indices.reshape(1, M)
    mesh = plsc.VectorSubcoreMesh(core_axis_name="core",
                                  subcore_axis_name="subcore")

    @pl.kernel(out_type=jax.ShapeDtypeStruct((M, W), table.dtype),
               mesh=mesh)
    def kern(x_hbm, i_hbm, o_hbm):
        def body(i_vmem, o_vmem):
            pltpu.sync_copy(x_hbm.at[i_vmem.at[0]], o_vmem)

        pltpu.emit_pipeline(
            body,
            grid=(M // GW,),
            in_specs=[pl.BlockSpec((1, GW), lambda i: (0, i))],
            out_specs=[pl.BlockSpec((GW, W), lambda i: (i, 0))],
            core_axis_name=("core", "subcore"),
            dimension_semantics=(pltpu.PARALLEL,),
        )(i_hbm, o_hbm)

    return kern(table, idx2).reshape(M // SPLIT, W0)


def _router_kernel(x_ref, rw_ref, po0_ref, po1_ref, va0_ref, va1_ref,
                   be_ref):
    xb = x_ref[...].astype(jnp.bfloat16)
    rwb = rw_ref[...].astype(jnp.bfloat16)
    logits = _dot(xb, rwb)
    m = jnp.max(logits, axis=-1, keepdims=True)
    p = jnp.exp(logits - m)
    p = p / jnp.sum(p, axis=-1, keepdims=True)
    iota = jax.lax.broadcasted_iota(jnp.int32, p.shape, 1)
    m1 = jnp.max(p, axis=-1, keepdims=True)
    i1 = jnp.min(jnp.where(p == m1, iota, E), axis=-1, keepdims=True)
    pm = jnp.where(iota == i1, -jnp.inf, p)
    m2 = jnp.max(pm, axis=-1, keepdims=True)
    i2 = jnp.min(jnp.where(pm == m2, iota, E), axis=-1, keepdims=True)
    va0_ref[...] = m1
    va1_ref[...] = m2
    oh1 = iota == i1
    oh2 = iota == i2
    ohb = (oh1 | oh2).astype(jnp.int32)
    incl = ohb
    d = 1
    while d < N:
        shifted = jnp.concatenate(
            [jnp.zeros((d, E), jnp.int32), incl[:N - d, :]], axis=0)
        incl = incl + shifted
        d *= 2
    excl = incl - ohb
    counts = incl[N - 1:N, :]
    pad_counts = ((counts + BLK - 1) // BLK) * BLK
    pad_end = pad_counts
    d = 1
    while d < E:
        pad_end = pad_end + jnp.concatenate(
            [jnp.zeros((1, d), jnp.int32), pad_end[:, :E - d]], axis=1)
        d *= 2
    pad_start = pad_end - pad_counts
    slot = excl + pad_start
    po0_ref[...] = jnp.sum(jnp.where(oh1, slot, 0), axis=1, keepdims=True)
    po1_ref[...] = jnp.sum(jnp.where(oh2, slot, 0), axis=1, keepdims=True)
    bb = BLK * jax.lax.broadcasted_iota(jnp.int32, (NBLK, E), 0)
    be = jnp.sum((jnp.broadcast_to(pad_end, (NBLK, E)) <= bb
                  ).astype(jnp.int32), axis=1, keepdims=True)
    nact = pad_end[:, E - 1:E] // BLK
    be_ref[...] = jnp.concatenate([jnp.minimum(be, E - 1), nact], axis=0)


def _shared_kernel(x_ref, gu_ref, dw_ref, sg_ref, sh_ref, gub_ref, dwb_ref):
    t = pl.program_id(0)

    @pl.when(t == 0)
    def _():
        gub_ref[...] = gu_ref[...].astype(jnp.bfloat16)
        dwb_ref[...] = dw_ref[...].astype(jnp.bfloat16)

    xb = x_ref[...].astype(jnp.bfloat16)
    gu = _dot(xb, gub_ref[...])
    h = (_silu(gu[:, :FF]) * gu[:, FF:]).astype(jnp.bfloat16)
    sh = _dot(h, dwb_ref[...])
    sgl = _dot(xb, sg_ref[...].astype(jnp.bfloat16))
    sh_ref[...] = sh * jax.nn.sigmoid(sgl)


def _gemm_kernel(be_ref, x_ref, po0_ref, po1_ref, va0_ref, va1_ref,
                 sh_ref, w1_ref, w2_ref, out_ref, xb_ref):
    b = pl.program_id(0)

    @pl.when(b == 0)
    def _():
        xb_ref[...] = x_ref[...].astype(jnp.bfloat16)
        out_ref[...] = sh_ref[...]

    @pl.when(b < be_ref[NBLK])
    def _():
        si = b * BLK + jax.lax.broadcasted_iota(jnp.int32, (BLK, N), 0)
        eq0 = po0_ref[...] == si
        eq1 = po1_ref[...] == si
        pm = (eq0 | eq1).astype(jnp.bfloat16)
        xs = _dot(pm, xb_ref[...]).astype(jnp.bfloat16)
        h = _dot(xs, w1_ref[0].astype(jnp.bfloat16))
        hb = _silu(h).astype(jnp.bfloat16)
        o = _dot(hb, w2_ref[0].astype(jnp.bfloat16)
                 ).astype(jnp.bfloat16)
        q = (jnp.where(eq0, va0_ref[...], 0.0) +
             jnp.where(eq1, va1_ref[...], 0.0)).astype(jnp.bfloat16)
        contrib = jax.lax.dot_general(q, o, (((0,), (0,)), ((), ())),
                                      preferred_element_type=jnp.float32)
        out_ref[...] += contrib


def kernel(x, router_w, w1, w2, gate_up_w, down_w, shared_gate_w):
    Bv, Tv, Dv = x.shape
    flat = x.reshape(N, D)

    po0, po1, va0, va1, blk_expert = pl.pallas_call(
        _router_kernel,
        grid=(1,),
        in_specs=[
            pl.BlockSpec((N, D), lambda t: (0, 0)),
            pl.BlockSpec((D, E), lambda t: (0, 0)),
        ],
        out_specs=[
            pl.BlockSpec((N, 1), lambda t: (0, 0)),
            pl.BlockSpec((N, 1), lambda t: (0, 0)),
            pl.BlockSpec((N, 1), lambda t: (0, 0)),
            pl.BlockSpec((N, 1), lambda t: (0, 0)),
            pl.BlockSpec((NBLK + 1, 1), lambda t: (0, 0)),
        ],
        out_shape=[
            jax.ShapeDtypeStruct((N, 1), jnp.int32),
            jax.ShapeDtypeStruct((N, 1), jnp.int32),
            jax.ShapeDtypeStruct((N, 1), jnp.float32),
            jax.ShapeDtypeStruct((N, 1), jnp.float32),
            jax.ShapeDtypeStruct((NBLK + 1, 1), jnp.int32),
        ],
    )(flat, router_w)

    sh = pl.pallas_call(
        _shared_kernel,
        grid=(N // BT,),
        in_specs=[
            pl.BlockSpec((BT, D), lambda t: (t, 0)),
            pl.BlockSpec((D, 2 * FF), lambda t: (0, 0)),
            pl.BlockSpec((FF, D), lambda t: (0, 0)),
            pl.BlockSpec((D, 1), lambda t: (0, 0)),
        ],
        out_specs=pl.BlockSpec((BT, D), lambda t: (t, 0)),
        out_shape=jax.ShapeDtypeStruct((N, D), jnp.float32),
        scratch_shapes=[
            pltpu.VMEM((D, 2 * FF), jnp.bfloat16),
            pltpu.VMEM((FF, D), jnp.bfloat16),
        ],
    )(flat, gate_up_w, down_w, shared_gate_w)

    out = pl.pallas_call(
        _gemm_kernel,
        grid_spec=pltpu.PrefetchScalarGridSpec(
            num_scalar_prefetch=1,
            grid=(NBLK,),
            in_specs=[
                pl.BlockSpec((N, D), lambda b, be: (0, 0)),
                pl.BlockSpec((1, N), lambda b, be: (0, 0)),
                pl.BlockSpec((1, N), lambda b, be: (0, 0)),
                pl.BlockSpec((1, N), lambda b, be: (0, 0)),
                pl.BlockSpec((1, N), lambda b, be: (0, 0)),
                pl.BlockSpec((N, D), lambda b, be: (0, 0)),
                pl.BlockSpec((1, D, FF), lambda b, be: (be[b], 0, 0)),
                pl.BlockSpec((1, FF, D), lambda b, be: (be[b], 0, 0)),
            ],
            out_specs=pl.BlockSpec((N, D), lambda b, be: (0, 0)),
            scratch_shapes=[pltpu.VMEM((N, D), jnp.bfloat16)],
        ),
        out_shape=jax.ShapeDtypeStruct((N, D), jnp.float32),
        compiler_params=pltpu.CompilerParams(
            dimension_semantics=("arbitrary",)),
    )(blk_expert.reshape(NBLK + 1), flat,
      po0.reshape(1, N), po1.reshape(1, N),
      va0.reshape(1, N), va1.reshape(1, N),
      sh, w1, w2)

    return out.reshape(Bv, Tv, Dv)


# speedup vs baseline: 1.0593x; 1.0593x over previous
import jax
import jax.numpy as jnp
from jax.experimental import pallas as pl
from jax.experimental.pallas import tpu as pltpu
from jax.experimental.pallas import tpu_sc as plsc

B, T, D = 1, 2048, 768
FF = 1536
E = 8
N = B * T
K = 2
NK = N * K

BLK = 256
NBLK = NK // BLK + E
PAD_N = NBLK * BLK
GW = 128
BT = 256


def _silu(v):
    return v * jax.nn.sigmoid(v)


def _dot(a, b):
    return jax.lax.dot_general(a, b, (((1,), (0,)), ((), ())),
                               preferred_element_type=jnp.float32)


def _sc_gather(table, indices):
    SPLIT = 2
    R, W0 = table.shape
    table = table.reshape(R * SPLIT, W0 // SPLIT)
    indices = (indices[:, None] * SPLIT +
               jnp.arange(SPLIT, dtype=jnp.int32)[None, :]).reshape(-1)
    M = indices.shape[0]
    W = table.shape[1]
    idx2 = indices.reshape(1, M)
    mesh = plsc.VectorSubcoreMesh(core_axis_name="core",
                                  subcore_axis_name="subcore")

    @pl.kernel(out_type=jax.ShapeDtypeStruct((M, W), table.dtype),
               mesh=mesh)
    def kern(x_hbm, i_hbm, o_hbm):
        def body(i_vmem, o_vmem):
            pltpu.sync_copy(x_hbm.at[i_vmem.at[0]], o_vmem)

        pltpu.emit_pipeline(
            body,
            grid=(M // GW,),
            in_specs=[pl.BlockSpec((1, GW), lambda i: (0, i))],
            out_specs=[pl.BlockSpec((GW, W), lambda i: (i, 0))],
            core_axis_name=("core", "subcore"),
            dimension_semantics=(pltpu.PARALLEL,),
        )(i_hbm, o_hbm)

    return kern(table, idx2).reshape(M // SPLIT, W0)


def _router_kernel(x_ref, rw_ref, gu_ref, dw_ref, sg_ref,
                   po0_ref, po1_ref, va0_ref, va1_ref, be_ref, sh_ref):
    gub = gu_ref[...].astype(jnp.bfloat16)
    dwb = dw_ref[...].astype(jnp.bfloat16)
    sgb = sg_ref[...].astype(jnp.bfloat16)
    for c in range(N // BT):
        xc = x_ref[c * BT:(c + 1) * BT, :].astype(jnp.bfloat16)
        gu = _dot(xc, gub)
        hsh = (_silu(gu[:, :FF]) * gu[:, FF:]).astype(jnp.bfloat16)
        shc = _dot(hsh, dwb)
        sgl = _dot(xc, sgb)
        sh_ref[c * BT:(c + 1) * BT, :] = shc * jax.nn.sigmoid(sgl)

    xb = x_ref[...].astype(jnp.bfloat16)
    rwb = rw_ref[...].astype(jnp.bfloat16)
    logits = _dot(xb, rwb)
    m = jnp.max(logits, axis=-1, keepdims=True)
    p = jnp.exp(logits - m)
    p = p / jnp.sum(p, axis=-1, keepdims=True)
    iota = jax.lax.broadcasted_iota(jnp.int32, p.shape, 1)
    m1 = jnp.max(p, axis=-1, keepdims=True)
    i1 = jnp.min(jnp.where(p == m1, iota, E), axis=-1, keepdims=True)
    pm = jnp.where(iota == i1, -jnp.inf, p)
    m2 = jnp.max(pm, axis=-1, keepdims=True)
    i2 = jnp.min(jnp.where(pm == m2, iota, E), axis=-1, keepdims=True)
    va0_ref[...] = m1
    va1_ref[...] = m2
    oh1 = iota == i1
    oh2 = iota == i2
    ohb = (oh1 | oh2).astype(jnp.int32)
    incl = ohb
    d = 1
    while d < N:
        shifted = jnp.concatenate(
            [jnp.zeros((d, E), jnp.int32), incl[:N - d, :]], axis=0)
        incl = incl + shifted
        d *= 2
    excl = incl - ohb
    counts = incl[N - 1:N, :]
    pad_counts = ((counts + BLK - 1) // BLK) * BLK
    pad_end = pad_counts
    d = 1
    while d < E:
        pad_end = pad_end + jnp.concatenate(
            [jnp.zeros((1, d), jnp.int32), pad_end[:, :E - d]], axis=1)
        d *= 2
    pad_start = pad_end - pad_counts
    slot = excl + pad_start
    po0_ref[...] = jnp.sum(jnp.where(oh1, slot, 0), axis=1, keepdims=True)
    po1_ref[...] = jnp.sum(jnp.where(oh2, slot, 0), axis=1, keepdims=True)
    bb = BLK * jax.lax.broadcasted_iota(jnp.int32, (NBLK, E), 0)
    be = jnp.sum((jnp.broadcast_to(pad_end, (NBLK, E)) <= bb
                  ).astype(jnp.int32), axis=1, keepdims=True)
    nact = pad_end[:, E - 1:E] // BLK
    be_ref[...] = jnp.concatenate([jnp.minimum(be, E - 1), nact], axis=0)


def _gemm_kernel(be_ref, x_ref, po0_ref, po1_ref, va0_ref, va1_ref,
                 sh_ref, w1_ref, w2_ref, out_ref, xb_ref, w1b_ref, w2b_ref):
    b = pl.program_id(0)

    @pl.when(b == 0)
    def _():
        xb_ref[...] = x_ref[...].astype(jnp.bfloat16)
        out_ref[...] = sh_ref[...]

    changed = jnp.logical_or(b == 0, be_ref[b] != be_ref[jnp.maximum(b - 1, 0)])

    @pl.when(jnp.logical_and(changed, b < be_ref[NBLK]))
    def _():
        w1b_ref[...] = w1_ref[0].astype(jnp.bfloat16)
        w2b_ref[...] = w2_ref[0].astype(jnp.bfloat16)

    @pl.when(b < be_ref[NBLK])
    def _():
        si = b * BLK + jax.lax.broadcasted_iota(jnp.int32, (BLK, N), 0)
        eq0 = po0_ref[...] == si
        eq1 = po1_ref[...] == si
        pm = (eq0 | eq1).astype(jnp.bfloat16)
        xs = _dot(pm, xb_ref[...]).astype(jnp.bfloat16)
        h = _dot(xs, w1b_ref[...])
        hb = _silu(h).astype(jnp.bfloat16)
        o = _dot(hb, w2b_ref[...]).astype(jnp.bfloat16)
        q = (jnp.where(eq0, va0_ref[...], 0.0) +
             jnp.where(eq1, va1_ref[...], 0.0)).astype(jnp.bfloat16)
        contrib = jax.lax.dot_general(q, o, (((0,), (0,)), ((), ())),
                                      preferred_element_type=jnp.float32)
        out_ref[...] += contrib


def kernel(x, router_w, w1, w2, gate_up_w, down_w, shared_gate_w):
    Bv, Tv, Dv = x.shape
    flat = x.reshape(N, D)

    po0, po1, va0, va1, blk_expert, sh = pl.pallas_call(
        _router_kernel,
        grid=(1,),
        in_specs=[
            pl.BlockSpec((N, D), lambda t: (0, 0)),
            pl.BlockSpec((D, E), lambda t: (0, 0)),
            pl.BlockSpec((D, 2 * FF), lambda t: (0, 0)),
            pl.BlockSpec((FF, D), lambda t: (0, 0)),
            pl.BlockSpec((D, 1), lambda t: (0, 0)),
        ],
        out_specs=[
            pl.BlockSpec((N, 1), lambda t: (0, 0)),
            pl.BlockSpec((N, 1), lambda t: (0, 0)),
            pl.BlockSpec((N, 1), lambda t: (0, 0)),
            pl.BlockSpec((N, 1), lambda t: (0, 0)),
            pl.BlockSpec((NBLK + 1, 1), lambda t: (0, 0)),
            pl.BlockSpec((N, D), lambda t: (0, 0)),
        ],
        out_shape=[
            jax.ShapeDtypeStruct((N, 1), jnp.int32),
            jax.ShapeDtypeStruct((N, 1), jnp.int32),
            jax.ShapeDtypeStruct((N, 1), jnp.float32),
            jax.ShapeDtypeStruct((N, 1), jnp.float32),
            jax.ShapeDtypeStruct((NBLK + 1, 1), jnp.int32),
            jax.ShapeDtypeStruct((N, D), jnp.float32),
        ],
    )(flat, router_w, gate_up_w, down_w, shared_gate_w)

    out = pl.pallas_call(
        _gemm_kernel,
        grid_spec=pltpu.PrefetchScalarGridSpec(
            num_scalar_prefetch=1,
            grid=(NBLK,),
            in_specs=[
                pl.BlockSpec((N, D), lambda b, be: (0, 0)),
                pl.BlockSpec((1, N), lambda b, be: (0, 0)),
                pl.BlockSpec((1, N), lambda b, be: (0, 0)),
                pl.BlockSpec((1, N), lambda b, be: (0, 0)),
                pl.BlockSpec((1, N), lambda b, be: (0, 0)),
                pl.BlockSpec((N, D), lambda b, be: (0, 0)),
                pl.BlockSpec((1, D, FF), lambda b, be: (be[b], 0, 0)),
                pl.BlockSpec((1, FF, D), lambda b, be: (be[b], 0, 0)),
            ],
            out_specs=pl.BlockSpec((N, D), lambda b, be: (0, 0)),
            scratch_shapes=[
                pltpu.VMEM((N, D), jnp.bfloat16),
                pltpu.VMEM((D, FF), jnp.bfloat16),
                pltpu.VMEM((FF, D), jnp.bfloat16),
            ],
        ),
        out_shape=jax.ShapeDtypeStruct((N, D), jnp.float32),
        compiler_params=pltpu.CompilerParams(
            dimension_semantics=("arbitrary",)),
    )(blk_expert.reshape(NBLK + 1), flat,
      po0.reshape(1, N), po1.reshape(1, N),
      va0.reshape(1, N), va1.reshape(1, N),
      sh, w1, w2)

    return out.reshape(Bv, Tv, Dv)


# speedup vs baseline: 1.0611x; 1.0017x over previous
import jax
import jax.numpy as jnp
from jax.experimental import pallas as pl
from jax.experimental.pallas import tpu as pltpu

B, T, D = 1, 2048, 768
FF = 1536
E = 8
N = B * T
K = 2
NK = N * K

BLK = 256
NBLK = NK // BLK + E
BT = 256


def _silu(v):
    return v * jax.nn.sigmoid(v)


def _dot(a, b):
    return jax.lax.dot_general(a, b, (((1,), (0,)), ((), ())),
                               preferred_element_type=jnp.float32)

def _router_kernel(x_ref, rw_ref, gu_ref, dw_ref, sg_ref,
                   po0_ref, po1_ref, va0_ref, va1_ref, be_ref, sh_ref):
    gub = gu_ref[...].astype(jnp.bfloat16)
    dwb = dw_ref[...].astype(jnp.bfloat16)
    sgb = sg_ref[...].astype(jnp.bfloat16)
    for c in range(N // BT):
        xc = x_ref[c * BT:(c + 1) * BT, :].astype(jnp.bfloat16)
        gu = _dot(xc, gub)
        hsh = (_silu(gu[:, :FF]) * gu[:, FF:]).astype(jnp.bfloat16)
        shc = _dot(hsh, dwb)
        sgl = _dot(xc, sgb)
        sh_ref[c * BT:(c + 1) * BT, :] = shc * jax.nn.sigmoid(sgl)

    xb = x_ref[...].astype(jnp.bfloat16)
    rwb = rw_ref[...].astype(jnp.bfloat16)
    logits = _dot(xb, rwb)
    m = jnp.max(logits, axis=-1, keepdims=True)
    p = jnp.exp(logits - m)
    p = p / jnp.sum(p, axis=-1, keepdims=True)
    iota = jax.lax.broadcasted_iota(jnp.int32, p.shape, 1)
    m1 = jnp.max(p, axis=-1, keepdims=True)
    i1 = jnp.min(jnp.where(p == m1, iota, E), axis=-1, keepdims=True)
    pm = jnp.where(iota == i1, -jnp.inf, p)
    m2 = jnp.max(pm, axis=-1, keepdims=True)
    i2 = jnp.min(jnp.where(pm == m2, iota, E), axis=-1, keepdims=True)
    va0_ref[...] = m1
    va1_ref[...] = m2
    oh1 = iota == i1
    oh2 = iota == i2
    ohb = (oh1 | oh2).astype(jnp.int32)
    incl = ohb
    d = 1
    while d < N:
        shifted = jnp.concatenate(
            [jnp.zeros((d, E), jnp.int32), incl[:N - d, :]], axis=0)
        incl = incl + shifted
        d *= 2
    excl = incl - ohb
    counts = incl[N - 1:N, :]
    pad_counts = ((counts + BLK - 1) // BLK) * BLK
    pad_end = pad_counts
    d = 1
    while d < E:
        pad_end = pad_end + jnp.concatenate(
            [jnp.zeros((1, d), jnp.int32), pad_end[:, :E - d]], axis=1)
        d *= 2
    pad_start = pad_end - pad_counts
    slot = excl + pad_start
    po0_ref[...] = jnp.sum(jnp.where(oh1, slot, 0), axis=1, keepdims=True)
    po1_ref[...] = jnp.sum(jnp.where(oh2, slot, 0), axis=1, keepdims=True)
    bb = BLK * jax.lax.broadcasted_iota(jnp.int32, (NBLK, E), 0)
    be = jnp.sum((jnp.broadcast_to(pad_end, (NBLK, E)) <= bb
                  ).astype(jnp.int32), axis=1, keepdims=True)
    nact = pad_end[:, E - 1:E] // BLK
    be_ref[...] = jnp.concatenate([jnp.minimum(be, E - 1), nact], axis=0)


def _gemm_kernel(be_ref, x_ref, po0_ref, po1_ref, va0_ref, va1_ref,
                 sh_ref, w1_ref, w2_ref, out_ref, xb_ref, w1b_ref, w2b_ref):
    b = pl.program_id(0)

    @pl.when(b == 0)
    def _():
        xb_ref[...] = x_ref[...].astype(jnp.bfloat16)
        out_ref[...] = sh_ref[...]

    changed = jnp.logical_or(b == 0, be_ref[b] != be_ref[jnp.maximum(b - 1, 0)])

    @pl.when(jnp.logical_and(changed, b < be_ref[NBLK]))
    def _():
        w1b_ref[...] = w1_ref[0].astype(jnp.bfloat16)
        w2b_ref[...] = w2_ref[0].astype(jnp.bfloat16)

    @pl.when(b < be_ref[NBLK])
    def _():
        si = b * BLK + jax.lax.broadcasted_iota(jnp.int32, (BLK, N), 0)
        eq0 = po0_ref[...] == si
        eq1 = po1_ref[...] == si
        pm = (eq0 | eq1).astype(jnp.bfloat16)
        xs = _dot(pm, xb_ref[...]).astype(jnp.bfloat16)
        h = _dot(xs, w1b_ref[...])
        hb = _silu(h).astype(jnp.bfloat16)
        o = _dot(hb, w2b_ref[...]).astype(jnp.bfloat16)
        q = (jnp.where(eq0, va0_ref[...], 0.0) +
             jnp.where(eq1, va1_ref[...], 0.0)).astype(jnp.bfloat16)
        contrib = jax.lax.dot_general(q, o, (((0,), (0,)), ((), ())),
                                      preferred_element_type=jnp.float32)
        out_ref[...] += contrib


def kernel(x, router_w, w1, w2, gate_up_w, down_w, shared_gate_w):
    Bv, Tv, Dv = x.shape
    flat = x.reshape(N, D)

    po0, po1, va0, va1, blk_expert, sh = pl.pallas_call(
        _router_kernel,
        grid=(1,),
        in_specs=[
            pl.BlockSpec((N, D), lambda t: (0, 0)),
            pl.BlockSpec((D, E), lambda t: (0, 0)),
            pl.BlockSpec((D, 2 * FF), lambda t: (0, 0)),
            pl.BlockSpec((FF, D), lambda t: (0, 0)),
            pl.BlockSpec((D, 1), lambda t: (0, 0)),
        ],
        out_specs=[
            pl.BlockSpec((N, 1), lambda t: (0, 0)),
            pl.BlockSpec((N, 1), lambda t: (0, 0)),
            pl.BlockSpec((N, 1), lambda t: (0, 0)),
            pl.BlockSpec((N, 1), lambda t: (0, 0)),
            pl.BlockSpec((NBLK + 1, 1), lambda t: (0, 0)),
            pl.BlockSpec((N, D), lambda t: (0, 0)),
        ],
        out_shape=[
            jax.ShapeDtypeStruct((N, 1), jnp.int32),
            jax.ShapeDtypeStruct((N, 1), jnp.int32),
            jax.ShapeDtypeStruct((N, 1), jnp.float32),
            jax.ShapeDtypeStruct((N, 1), jnp.float32),
            jax.ShapeDtypeStruct((NBLK + 1, 1), jnp.int32),
            jax.ShapeDtypeStruct((N, D), jnp.float32),
        ],
    )(flat, router_w, gate_up_w, down_w, shared_gate_w)

    out = pl.pallas_call(
        _gemm_kernel,
        grid_spec=pltpu.PrefetchScalarGridSpec(
            num_scalar_prefetch=1,
            grid=(NBLK,),
            in_specs=[
                pl.BlockSpec((N, D), lambda b, be: (0, 0)),
                pl.BlockSpec((1, N), lambda b, be: (0, 0)),
                pl.BlockSpec((1, N), lambda b, be: (0, 0)),
                pl.BlockSpec((1, N), lambda b, be: (0, 0)),
                pl.BlockSpec((1, N), lambda b, be: (0, 0)),
                pl.BlockSpec((N, D), lambda b, be: (0, 0)),
                pl.BlockSpec((1, D, FF), lambda b, be: (be[b], 0, 0)),
                pl.BlockSpec((1, FF, D), lambda b, be: (be[b], 0, 0)),
            ],
            out_specs=pl.BlockSpec((N, D), lambda b, be: (0, 0)),
            scratch_shapes=[
                pltpu.VMEM((N, D), jnp.bfloat16),
                pltpu.VMEM((D, FF), jnp.bfloat16),
                pltpu.VMEM((FF, D), jnp.bfloat16),
            ],
        ),
        out_shape=jax.ShapeDtypeStruct((N, D), jnp.float32),
        compiler_params=pltpu.CompilerParams(
            dimension_semantics=("arbitrary",)),
    )(blk_expert.reshape(NBLK + 1), flat,
      po0.reshape(1, N), po1.reshape(1, N),
      va0.reshape(1, N), va1.reshape(1, N),
      sh, w1, w2)

    return out.reshape(Bv, Tv, Dv)
